# baseline (speedup 1.0000x reference)
import jax
import jax.numpy as jnp
from jax import lax
from jax.experimental import pallas as pl
from jax.experimental.pallas import tpu as pltpu
from jax.experimental.pallas import tpu_sc as plsc

N = 100000
V = 100000
D_A = 32
D_R = 16
D_N = 16
D_OUT = D_A + D_R + D_N

NW = 32
CB = 256
CPW = 13
LAST = N - CB
NBUF = 4
L = 16


def _body(at_hbm, rt_hbm, ext_hbm, wa_hbm, wr_hbm, out_hbm, *scr):
    idx_a = scr[0:NBUF]
    idx_r = scr[NBUF:2 * NBUF]
    rows_a = scr[2 * NBUF:3 * NBUF]
    rows_r = scr[3 * NBUF:4 * NBUF]
    out_v = scr[4 * NBUF:5 * NBUF]
    sem_i = scr[5 * NBUF:6 * NBUF]
    sem_g = scr[6 * NBUF:7 * NBUF]
    sem_e = scr[7 * NBUF:8 * NBUF]
    sem_s = scr[8 * NBUF:9 * NBUF]

    wid = lax.axis_index("s") * 2 + lax.axis_index("c")
    offs = [jnp.minimum((wid * CPW + t) * CB, LAST) for t in range(CPW)]
    d = {}

    def stage_a(t):
        p = t % NBUF
        d["ia", t] = pltpu.async_copy(at_hbm.at[pl.ds(offs[t], CB)], idx_a[p], sem_i[p])
        d["ir", t] = pltpu.async_copy(rt_hbm.at[pl.ds(offs[t], CB)], idx_r[p], sem_i[p])

    def stage_b(t):
        p = t % NBUF
        d["ia", t].wait()
        d["ir", t].wait()
        d["ga", t] = pltpu.async_copy(wa_hbm.at[idx_a[p]], rows_a[p], sem_g[p])
        d["gr", t] = pltpu.async_copy(wr_hbm.at[idx_r[p]], rows_r[p], sem_g[p])
        d["e", t] = pltpu.async_copy(
            ext_hbm.at[:, pl.ds(offs[t], CB)], out_v[p].at[pl.ds(D_A + D_R, D_N), :], sem_e[p])

    def stage_x(t):
        p = t % NBUF
        d["ga", t].wait()
        d["gr", t].wait()

        def group(g, carry):
            rvec = g * L + lax.iota(jnp.int32, L)
            for j in range(D_A):
                out_v[p][j, pl.ds(g * L, L)] = plsc.load_gather(
                    rows_a[p], [rvec, jnp.full((L,), j, jnp.int32)])
            for j in range(D_R):
                out_v[p][D_A + j, pl.ds(g * L, L)] = plsc.load_gather(
                    rows_r[p], [rvec, jnp.full((L,), j, jnp.int32)])
            return carry

        lax.fori_loop(0, CB // L, group, 0)

    def stage_s(t):
        p = t % NBUF
        d["e", t].wait()
        d["s", t] = pltpu.async_copy(out_v[p], out_hbm.at[:, pl.ds(offs[t], CB)], sem_s[p])

    def drain(t):
        d["s", t].wait()

    for t in range(CPW + 3):
        if t < CPW:
            if t >= NBUF:
                drain(t - NBUF)
            stage_a(t)
        if 1 <= t and t - 1 < CPW:
            stage_b(t - 1)
        if 2 <= t and t - 2 < CPW:
            stage_x(t - 2)
        if 3 <= t and t - 3 < CPW:
            stage_s(t - 3)
    for t in range(max(0, CPW - NBUF), CPW):
        drain(t)


@jax.jit
def _run(atom_types, residue_types, extra_feats, W_atom, W_res):
    mesh = plsc.VectorSubcoreMesh(core_axis_name="c", subcore_axis_name="s")
    scratch = (
        [pltpu.VMEM((CB,), jnp.int32) for _ in range(NBUF)]
        + [pltpu.VMEM((CB,), jnp.int32) for _ in range(NBUF)]
        + [pltpu.VMEM((CB, D_A), jnp.float32) for _ in range(NBUF)]
        + [pltpu.VMEM((CB, D_R), jnp.float32) for _ in range(NBUF)]
        + [pltpu.VMEM((D_OUT, CB), jnp.float32) for _ in range(NBUF)]
        + [pltpu.SemaphoreType.DMA for _ in range(4 * NBUF)]
    )
    f = pl.kernel(
        _body,
        mesh=mesh,
        compiler_params=pltpu.CompilerParams(
            use_tc_tiling_on_sc=False, needs_layout_passes=False),
        out_type=jax.ShapeDtypeStruct((D_OUT, N), jnp.float32),
        scratch_types=scratch,
    )
    out_t = f(atom_types, residue_types, extra_feats.T, W_atom, W_res)
    return out_t.T


def kernel(atom_types, residue_types, extra_feats, W_atom, W_res):
    return _run(atom_types, residue_types, extra_feats, W_atom, W_res)
